# baseline (device time: 1184094 ns/iter reference)
import jax
import jax.numpy as jnp
from jax import lax
from jax.experimental import pallas as pl
from jax.experimental.pallas import tpu as pltpu


def kernel(x):
    m, n = x.shape
    half = n // 2

    x16 = x.astype(jnp.bfloat16)
    my_z = lax.axis_index("z")
    mine = lax.dynamic_slice_in_dim(x16, my_z * half, half, axis=1)
    theirs = lax.dynamic_slice_in_dim(x16, (1 - my_z) * half, half, axis=1)

    N_LOCAL_CHUNKS = 16

    def body(mine_ref, theirs_ref, out_ref, local_sem, send_sem, recv_sem):
        mx = lax.axis_index("x")
        my = lax.axis_index("y")
        mz = lax.axis_index("z")
        peer = (mx, my, 1 - mz)

        barrier = pltpu.get_barrier_semaphore()
        pl.semaphore_signal(
            barrier, inc=1,
            device_id=peer,
            device_id_type=pl.DeviceIdType.MESH,
        )
        pl.semaphore_wait(barrier, 1)

        rdma = pltpu.make_async_remote_copy(
            src_ref=theirs_ref,
            dst_ref=out_ref.at[pl.ds(mz * m, m), :],
            send_sem=send_sem,
            recv_sem=recv_sem,
            device_id=peer,
            device_id_type=pl.DeviceIdType.MESH,
        )
        rdma.start()

        chunk = m // N_LOCAL_CHUNKS
        locals_ = []
        for i in range(N_LOCAL_CHUNKS):
            c = pltpu.make_async_copy(
                mine_ref.at[pl.ds(i * chunk, chunk), :],
                out_ref.at[pl.ds(mz * m + i * chunk, chunk), :],
                local_sem,
            )
            c.start()
            locals_.append(c)
        for c in locals_:
            c.wait()

        rdma.wait()

    return pl.pallas_call(
        body,
        out_shape=jax.ShapeDtypeStruct((2 * m, half), jnp.bfloat16),
        in_specs=[
            pl.BlockSpec(memory_space=pl.ANY),
            pl.BlockSpec(memory_space=pl.ANY),
        ],
        out_specs=pl.BlockSpec(memory_space=pl.ANY),
        scratch_shapes=[
            pltpu.SemaphoreType.DMA,
            pltpu.SemaphoreType.DMA,
            pltpu.SemaphoreType.DMA,
        ],
        compiler_params=pltpu.CompilerParams(collective_id=0),
    )(mine, theirs)


# device time: 409443 ns/iter; 2.8920x vs baseline; 2.8920x over previous
import jax
import jax.numpy as jnp
from jax import lax
from jax.experimental import pallas as pl
from jax.experimental.pallas import tpu as pltpu

N_CHUNKS = 16
N_STAGE = 2


def kernel(x):
    m, n = x.shape
    half = n // 2
    cr = m // N_CHUNKS

    def body(x_ref, out_ref,
             stage_s, send_buf, stage_l, local_buf,
             stage_s_sem, stage_l_sem, local_out_sem,
             send_sems, recv_sems):
        mx = lax.axis_index("x")
        my = lax.axis_index("y")
        mz = lax.axis_index("z")
        peer = (mx, my, 1 - mz)
        my_col = mz * half
        peer_col = (1 - mz) * half

        barrier = pltpu.get_barrier_semaphore()
        pl.semaphore_signal(
            barrier, inc=1,
            device_id=peer,
            device_id_type=pl.DeviceIdType.MESH,
        )
        pl.semaphore_wait(barrier, 1)

        def load(i, col0, stage, sem):
            c = pltpu.make_async_copy(
                x_ref.at[pl.ds(i * cr, cr), pl.ds(col0, half)],
                stage.at[i % N_STAGE],
                sem.at[i % N_STAGE],
            )
            c.start()
            return c

        rdmas = []
        loads = {i: load(i, peer_col, stage_s, stage_s_sem)
                 for i in range(N_STAGE)}
        for i in range(N_CHUNKS):
            loads[i].wait()
            send_buf[i, :, :] = stage_s[i % N_STAGE].astype(jnp.bfloat16)
            if i + N_STAGE < N_CHUNKS:
                loads[i + N_STAGE] = load(
                    i + N_STAGE, peer_col, stage_s, stage_s_sem)
            r = pltpu.make_async_remote_copy(
                src_ref=send_buf.at[i],
                dst_ref=out_ref.at[pl.ds(mz * m + i * cr, cr), :],
                send_sem=send_sems.at[i],
                recv_sem=recv_sems.at[i],
                device_id=peer,
                device_id_type=pl.DeviceIdType.MESH,
            )
            r.start()
            rdmas.append(r)

        lloads = {i: load(i, my_col, stage_l, stage_l_sem)
                  for i in range(N_STAGE)}
        lstores = {}
        for i in range(N_CHUNKS):
            lloads[i].wait()
            if i >= N_STAGE:
                lstores[i - N_STAGE].wait()
            local_buf[i % N_STAGE, :, :] = (
                stage_l[i % N_STAGE].astype(jnp.bfloat16))
            if i + N_STAGE < N_CHUNKS:
                lloads[i + N_STAGE] = load(
                    i + N_STAGE, my_col, stage_l, stage_l_sem)
            st = pltpu.make_async_copy(
                local_buf.at[i % N_STAGE],
                out_ref.at[pl.ds(mz * m + i * cr, cr), :],
                local_out_sem.at[i % N_STAGE],
            )
            st.start()
            lstores[i] = st
        for i in range(N_CHUNKS - N_STAGE, N_CHUNKS):
            lstores[i].wait()

        for r in rdmas:
            r.wait()

    return pl.pallas_call(
        body,
        out_shape=jax.ShapeDtypeStruct((2 * m, half), jnp.bfloat16),
        in_specs=[pl.BlockSpec(memory_space=pl.ANY)],
        out_specs=pl.BlockSpec(memory_space=pl.ANY),
        scratch_shapes=[
            pltpu.VMEM((N_STAGE, cr, half), jnp.float32),
            pltpu.VMEM((N_CHUNKS, cr, half), jnp.bfloat16),
            pltpu.VMEM((N_STAGE, cr, half), jnp.float32),
            pltpu.VMEM((N_STAGE, cr, half), jnp.bfloat16),
            pltpu.SemaphoreType.DMA((N_STAGE,)),
            pltpu.SemaphoreType.DMA((N_STAGE,)),
            pltpu.SemaphoreType.DMA((N_STAGE,)),
            pltpu.SemaphoreType.DMA((N_CHUNKS,)),
            pltpu.SemaphoreType.DMA((N_CHUNKS,)),
        ],
        compiler_params=pltpu.CompilerParams(
            collective_id=0,
            vmem_limit_bytes=60 * 1024 * 1024,
        ),
    )(x)
